# baseline (speedup 1.0000x reference)
import functools

import jax
import jax.numpy as jnp
from jax import lax
from jax.experimental import pallas as pl
from jax.experimental.pallas import tpu as pltpu

NHEAD = 4
H = 1024
DH = H // NHEAD
BLK = 512
_QSCALE = 1.4426950408889634 / (DH ** 0.5)


def _dot_t(a, w):
    return lax.dot_general(a, w, (((1,), (1,)), ((), ())),
                           preferred_element_type=jnp.float32)


def _dot(a, w):
    return jnp.dot(a, w, preferred_element_type=jnp.float32)


def _dot_t_fast(a, w_bf16):
    return lax.dot_general(a.astype(jnp.bfloat16), w_bf16,
                           (((1,), (1,)), ((), ())),
                           preferred_element_type=jnp.float32)


def _route_body(xrow_ref, wg_ref, bg_ref, be_ref, we_hbm,
                wf_ref, bfv_ref, wcd_ref, bcd_ref,
                wk2_ref, bk2_ref, wout_ref, bout_ref,
                wc_ref, bc_ref, wa_ref, ba_ref, wb_ref, bb_ref,
                w_a, w_b, sem_a, sem_b, xrow_scr):
    i = pl.program_id(0)

    @pl.when(i == 0)
    def _():
        xrow_scr[...] = xrow_ref[...]
        wa_ref[...] = _dot(wcd_ref[...], wf_ref[...])
        ba_ref[...] = _dot_t(bfv_ref[...], wcd_ref[...]) + bcd_ref[...]
        wb_ref[...] = _dot(wout_ref[...], wk2_ref[...])
        bb_ref[...] = _dot_t(bk2_ref[...], wout_ref[...]) + bout_ref[...]

    xr = xrow_scr[...]
    logits = _dot_t(xr, wg_ref[i]) + bg_ref[i]
    iota = lax.broadcasted_iota(jnp.int32, logits.shape, 1)
    m1 = jnp.max(logits)
    i1 = jnp.min(jnp.where(logits >= m1, iota, 8)).astype(jnp.int32)
    masked = jnp.where(iota == i1, -jnp.inf, logits)
    m2 = jnp.max(masked)
    i2 = jnp.min(jnp.where(masked >= m2, iota, 8)).astype(jnp.int32)

    cp_a = pltpu.make_async_copy(we_hbm.at[i, i1], w_a, sem_a)
    cp_b = pltpu.make_async_copy(we_hbm.at[i, i2], w_b, sem_b)
    cp_a.start()
    cp_b.start()
    cp_a.wait()
    cp_b.wait()

    wavg = 0.5 * (w_a[...] + w_b[...])
    bavg = 0.5 * (be_ref[i, i1] + be_ref[i, i2])

    @pl.when(i == 0)
    def _():
        wc_ref[...] = wavg
        bc_ref[...] = bavg

    @pl.when(i > 0)
    def _():
        wc_ref[...] = _dot(wavg, wc_ref[...])
        bc_ref[...] = _dot_t(bc_ref[...], wavg) + bavg

    xrow_scr[...] = _dot_t(xr, wavg) + bavg


def _route(xrow, wg_all, bg_all, be_all, we_all,
           wf, bfv, wcd, bcd, wk2, bk2, wout, bout):
    wspec = pl.BlockSpec((H, H), lambda i: (0, 0))
    bspec = pl.BlockSpec((1, H), lambda i: (0, 0))
    return pl.pallas_call(
        _route_body,
        grid=(3,),
        in_specs=[
            pl.BlockSpec((1, H), lambda i: (0, 0)),
            pl.BlockSpec((3, 8, H), lambda i: (0, 0, 0)),
            pl.BlockSpec((3, 1, 8), lambda i: (0, 0, 0)),
            pl.BlockSpec((3, 8, 1, H), lambda i: (0, 0, 0, 0)),
            pl.BlockSpec(memory_space=pl.ANY),
            wspec, bspec, wspec, bspec,
            wspec, bspec, wspec, bspec,
        ],
        out_specs=[
            wspec, bspec, wspec, bspec, wspec, bspec,
        ],
        out_shape=[
            jax.ShapeDtypeStruct((H, H), jnp.float32),
            jax.ShapeDtypeStruct((1, H), jnp.float32),
            jax.ShapeDtypeStruct((H, H), jnp.float32),
            jax.ShapeDtypeStruct((1, H), jnp.float32),
            jax.ShapeDtypeStruct((H, H), jnp.float32),
            jax.ShapeDtypeStruct((1, H), jnp.float32),
        ],
        scratch_shapes=[
            pltpu.VMEM((H, H), jnp.float32),
            pltpu.VMEM((H, H), jnp.float32),
            pltpu.SemaphoreType.DMA,
            pltpu.SemaphoreType.DMA,
            pltpu.VMEM((1, H), jnp.float32),
        ],
    )(xrow, wg_all, bg_all, be_all, we_all,
      wf, bfv, wcd, bcd, wk2, bk2, wout, bout)


def _gate_body(xrow_ref, wg_ref, bg_ref, idx_ref):
    logits = _dot_t(xrow_ref[...], wg_ref[0]) + bg_ref[0]
    iota = lax.broadcasted_iota(jnp.int32, logits.shape, 1)
    m1 = jnp.max(logits)
    i1 = jnp.min(jnp.where(logits >= m1, iota, 8))
    masked = jnp.where(iota == i1, -jnp.inf, logits)
    m2 = jnp.max(masked)
    i2 = jnp.min(jnp.where(masked >= m2, iota, 8))
    idx_ref[0] = i1.astype(jnp.int32)
    idx_ref[1] = i2.astype(jnp.int32)


def _gate(xrow, wg_all, bg_all, layer):
    return pl.pallas_call(
        _gate_body,
        grid=(1,),
        in_specs=[
            pl.BlockSpec((1, H), lambda t: (0, 0)),
            pl.BlockSpec((1, 8, H), lambda t: (layer, 0, 0)),
            pl.BlockSpec((1, 1, 8), lambda t: (layer, 0, 0)),
        ],
        out_shape=jax.ShapeDtypeStruct((2,), jnp.int32),
        out_specs=pl.BlockSpec(memory_space=pltpu.SMEM),
    )(xrow, wg_all, bg_all)


def _avg_body(idx_ref, we_ref, be_ref, xrow_ref, wavg_ref, bavg_ref,
              xnext_ref):
    k = pl.program_id(0)

    @pl.when(k == 0)
    def _():
        wavg_ref[...] = jnp.zeros_like(wavg_ref)
        bavg_ref[...] = jnp.zeros_like(bavg_ref)
        xnext_ref[...] = jnp.zeros_like(xnext_ref)

    w = we_ref[0, 0]
    b = be_ref[0, 0]
    wavg_ref[...] += 0.5 * w
    bavg_ref[...] += 0.5 * b
    xnext_ref[...] += 0.5 * (_dot_t(xrow_ref[...], w) + b)


def _avg(idx, we_all, be_all, xrow, layer):
    grid_spec = pltpu.PrefetchScalarGridSpec(
        num_scalar_prefetch=1,
        grid=(2,),
        in_specs=[
            pl.BlockSpec((1, 1, H, H),
                         lambda k, idx_ref: (layer, idx_ref[k], 0, 0)),
            pl.BlockSpec((1, 1, 1, H),
                         lambda k, idx_ref: (layer, idx_ref[k], 0, 0)),
            pl.BlockSpec((1, H), lambda k, idx_ref: (0, 0)),
        ],
        out_specs=[
            pl.BlockSpec((H, H), lambda k, idx_ref: (0, 0)),
            pl.BlockSpec((1, H), lambda k, idx_ref: (0, 0)),
            pl.BlockSpec((1, H), lambda k, idx_ref: (0, 0)),
        ],
    )
    return pl.pallas_call(
        _avg_body,
        grid_spec=grid_spec,
        out_shape=[
            jax.ShapeDtypeStruct((H, H), jnp.float32),
            jax.ShapeDtypeStruct((1, H), jnp.float32),
            jax.ShapeDtypeStruct((1, H), jnp.float32),
        ],
    )(idx, we_all, be_all, xrow)


def _compose_body(w0_ref, b0_ref, w1_ref, b1_ref, w2_ref, b2_ref,
                  wc_ref, bc_ref):
    w21 = _dot(w2_ref[...], w1_ref[...])
    wc_ref[...] = _dot(w21, w0_ref[...])
    b01 = _dot_t(b0_ref[...], w1_ref[...]) + b1_ref[...]
    bc_ref[...] = _dot_t(b01, w2_ref[...]) + b2_ref[...]


def _compose(w0, b0, w1, b1, w2, b2):
    return pl.pallas_call(
        _compose_body,
        out_shape=[
            jax.ShapeDtypeStruct((H, H), jnp.float32),
            jax.ShapeDtypeStruct((1, H), jnp.float32),
        ],
    )(w0, b0, w1, b1, w2, b2)


def _moe_qkv_body(x_ref, wc_ref, bc_ref, wqkv_ref, bqkv_ref,
                  xmoe_ref, qkv_ref):
    y = _dot_t(x_ref[...], wc_ref[...]) + bc_ref[...]
    xmoe_ref[...] = y
    z = _dot_t(y, wqkv_ref[...]) + bqkv_ref[...]
    zq = z[:, :H] * _QSCALE
    qkv_ref[:, :H] = zq.astype(jnp.bfloat16)
    qkv_ref[:, H:] = z[:, H:].astype(jnp.bfloat16)


def _moe_qkv(x, wc, bc, wqkv, bqkv):
    S = x.shape[0]
    full = lambda t: (0, 0)
    return pl.pallas_call(
        _moe_qkv_body,
        grid=(S // BLK,),
        in_specs=[
            pl.BlockSpec((BLK, H), lambda t: (t, 0)),
            pl.BlockSpec((H, H), full), pl.BlockSpec((1, H), full),
            pl.BlockSpec((3 * H, H), full), pl.BlockSpec((1, 3 * H), full),
        ],
        out_specs=[
            pl.BlockSpec((BLK, H), lambda t: (t, 0)),
            pl.BlockSpec((BLK, 3 * H), lambda t: (t, 0)),
        ],
        out_shape=[
            jax.ShapeDtypeStruct((S, H), jnp.float32),
            jax.ShapeDtypeStruct((S, 3 * H), jnp.bfloat16),
        ],
        compiler_params=pltpu.CompilerParams(
            dimension_semantics=("parallel",)),
    )(x, wc, bc, wqkv, bqkv)


BQ = 1024


def _attn_body(q_ref, k_ref, v_ref, o_ref):
    q = q_ref[...]
    k = k_ref[...]
    s = lax.dot_general(q, k, (((1,), (1,)), ((), ())),
                        preferred_element_type=jnp.float32)
    e = jnp.exp2(s)
    r = 1.0 / jnp.sum(e, axis=-1, keepdims=True)
    o_ref[...] = jnp.dot(e.astype(jnp.bfloat16), v_ref[...],
                         preferred_element_type=jnp.float32) * r


def _attention(qkv, S):
    return pl.pallas_call(
        _attn_body,
        grid=(NHEAD, S // BQ),
        in_specs=[
            pl.BlockSpec((BQ, DH), lambda h, t: (t, h)),
            pl.BlockSpec((S, DH), lambda h, t: (0, NHEAD + h)),
            pl.BlockSpec((S, DH), lambda h, t: (0, 2 * NHEAD + h)),
        ],
        out_specs=pl.BlockSpec((BQ, DH), lambda h, t: (t, h)),
        out_shape=jax.ShapeDtypeStruct((S, H), jnp.float32),
        compiler_params=pltpu.CompilerParams(
            dimension_semantics=("parallel", "parallel")),
    )(qkv, qkv, qkv)


def _ln(x, g, b):
    m = jnp.mean(x, axis=-1, keepdims=True)
    c = x - m
    v = jnp.mean(c * c, axis=-1, keepdims=True)
    return c * lax.rsqrt(v + 1e-5) * g + b


def _post_body(xmoe_ref, ao_ref, wo_ref, bo_ref, g1_ref, be1_ref,
               w1_ref, b1_ref, w2_ref, b2_ref, g2_ref, be2_ref,
               wa_ref, ba_ref, wk1_ref, bk1_ref, wb_ref, bb_ref, o_ref):
    t = _dot_t(ao_ref[...], wo_ref[...]) + bo_ref[...]
    x = _ln(xmoe_ref[...] + t, g1_ref[...], be1_ref[...])
    h = jnp.maximum(_dot_t(x, w1_ref[...]) + b1_ref[...], 0.0)
    f = _dot_t(h, w2_ref[...]) + b2_ref[...]
    x = _ln(x + f, g2_ref[...], be2_ref[...])
    x = _dot_t(x, wa_ref[...]) + ba_ref[...]
    h = jnp.maximum(_dot_t(x, wk1_ref[...]) + bk1_ref[...], 0.0)
    o_ref[...] = _dot_t(h, wb_ref[...]) + bb_ref[...]


def _post_tail(xmoe, ao, wo, bo, g1, be1, w1, b1, w2, b2, g2, be2,
               wa, ba, wk1, bk1, wb, bb):
    S = xmoe.shape[0]
    full = lambda t: (0, 0)
    wspec = pl.BlockSpec((H, H), full)
    bspec = pl.BlockSpec((1, H), full)
    blk = pl.BlockSpec((BLK, H), lambda t: (t, 0))
    return pl.pallas_call(
        _post_body,
        grid=(S // BLK,),
        in_specs=[
            blk, blk,
            wspec, bspec, bspec, bspec,
            pl.BlockSpec((2 * H, H), full), pl.BlockSpec((1, 2 * H), full),
            pl.BlockSpec((H, 2 * H), full), bspec,
            bspec, bspec,
            wspec, bspec, wspec, bspec, wspec, bspec,
        ],
        out_specs=blk,
        out_shape=jax.ShapeDtypeStruct((S, H), jnp.float32),
        compiler_params=pltpu.CompilerParams(
            dimension_semantics=("parallel",)),
    )(xmoe, ao, wo, bo, g1, be1, w1, b1, w2, b2, g2, be2,
      wa, ba, wk1, bk1, wb, bb)


def kernel(x, moe_We, moe_be, moe_Wg, moe_bg, Wqkv, bqkv, Wo, bo, ln1_g,
           ln1_b, W1, b1, W2, b2, ln2_g, ln2_b, Wf, bf, Wc, bc, Wk1, bk1,
           Wk2, bk2, Wout, bout):
    S = x.shape[0]
    row = lambda v: v.reshape(1, -1)

    wc_moe, bc_moe, wa, ba, wb, bb = _route(
        x[0:1], moe_Wg, moe_bg.reshape(3, 1, 8),
        moe_be.reshape(3, 8, 1, H), moe_We,
        Wf, row(bf), Wc, row(bc), Wk2, row(bk2), Wout, row(bout))

    xmoe, qkv = _moe_qkv(x, wc_moe, bc_moe, Wqkv, row(bqkv))
    ao = _attention(qkv, S)
    return _post_tail(xmoe, ao, Wo, row(bo), row(ln1_g), row(ln1_b),
                      W1, row(b1), W2, row(b2), row(ln2_g), row(ln2_b),
                      wa, ba, Wk1, row(bk1), wb, bb)


# speedup vs baseline: 1.0121x; 1.0121x over previous
import functools

import jax
import jax.numpy as jnp
from jax import lax
from jax.experimental import pallas as pl
from jax.experimental.pallas import tpu as pltpu

NHEAD = 4
H = 1024
DH = H // NHEAD
BLK = 512
_QSCALE = 1.4426950408889634 / (DH ** 0.5)


def _dot_t(a, w):
    return lax.dot_general(a, w, (((1,), (1,)), ((), ())),
                           preferred_element_type=jnp.float32)


def _dot(a, w):
    return jnp.dot(a, w, preferred_element_type=jnp.float32)


def _dot_t_fast(a, w_bf16):
    return lax.dot_general(a.astype(jnp.bfloat16), w_bf16,
                           (((1,), (1,)), ((), ())),
                           preferred_element_type=jnp.float32)


def _route_body(xrow_ref, wg_ref, bg_ref, be_ref, we_hbm,
                wf_ref, bfv_ref, wcd_ref, bcd_ref,
                wk2_ref, bk2_ref, wout_ref, bout_ref,
                wc_ref, bc_ref, wa_ref, ba_ref, wb_ref, bb_ref,
                w_a, w_b, sem_a, sem_b, xrow_scr):
    i = pl.program_id(0)

    @pl.when(i == 0)
    def _():
        xrow_scr[...] = xrow_ref[...]
        wa_ref[...] = _dot(wcd_ref[...], wf_ref[...])
        ba_ref[...] = _dot_t(bfv_ref[...], wcd_ref[...]) + bcd_ref[...]
        wb_ref[...] = _dot(wout_ref[...], wk2_ref[...])
        bb_ref[...] = _dot_t(bk2_ref[...], wout_ref[...]) + bout_ref[...]

    xr = xrow_scr[...]
    logits = _dot_t(xr, wg_ref[i]) + bg_ref[i]
    iota = lax.broadcasted_iota(jnp.int32, logits.shape, 1)
    m1 = jnp.max(logits)
    i1 = jnp.min(jnp.where(logits >= m1, iota, 8)).astype(jnp.int32)
    masked = jnp.where(iota == i1, -jnp.inf, logits)
    m2 = jnp.max(masked)
    i2 = jnp.min(jnp.where(masked >= m2, iota, 8)).astype(jnp.int32)

    cp_a = pltpu.make_async_copy(we_hbm.at[i, i1], w_a, sem_a)
    cp_b = pltpu.make_async_copy(we_hbm.at[i, i2], w_b, sem_b)
    cp_a.start()
    cp_b.start()
    cp_a.wait()
    cp_b.wait()

    wavg = 0.5 * (w_a[...] + w_b[...])
    bavg = 0.5 * (be_ref[i, i1] + be_ref[i, i2])

    @pl.when(i == 0)
    def _():
        wc_ref[...] = wavg
        bc_ref[...] = bavg

    @pl.when(i > 0)
    def _():
        wc_ref[...] = _dot(wavg, wc_ref[...])
        bc_ref[...] = _dot_t(bc_ref[...], wavg) + bavg

    xrow_scr[...] = _dot_t(xr, wavg) + bavg


def _route(xrow, wg_all, bg_all, be_all, we_all,
           wf, bfv, wcd, bcd, wk2, bk2, wout, bout):
    wspec = pl.BlockSpec((H, H), lambda i: (0, 0))
    bspec = pl.BlockSpec((1, H), lambda i: (0, 0))
    return pl.pallas_call(
        _route_body,
        grid=(3,),
        in_specs=[
            pl.BlockSpec((1, H), lambda i: (0, 0)),
            pl.BlockSpec((3, 8, H), lambda i: (0, 0, 0)),
            pl.BlockSpec((3, 1, 8), lambda i: (0, 0, 0)),
            pl.BlockSpec((3, 8, 1, H), lambda i: (0, 0, 0, 0)),
            pl.BlockSpec(memory_space=pl.ANY),
            wspec, bspec, wspec, bspec,
            wspec, bspec, wspec, bspec,
        ],
        out_specs=[
            wspec, bspec, wspec, bspec, wspec, bspec,
        ],
        out_shape=[
            jax.ShapeDtypeStruct((H, H), jnp.float32),
            jax.ShapeDtypeStruct((1, H), jnp.float32),
            jax.ShapeDtypeStruct((H, H), jnp.float32),
            jax.ShapeDtypeStruct((1, H), jnp.float32),
            jax.ShapeDtypeStruct((H, H), jnp.float32),
            jax.ShapeDtypeStruct((1, H), jnp.float32),
        ],
        scratch_shapes=[
            pltpu.VMEM((H, H), jnp.float32),
            pltpu.VMEM((H, H), jnp.float32),
            pltpu.SemaphoreType.DMA,
            pltpu.SemaphoreType.DMA,
            pltpu.VMEM((1, H), jnp.float32),
        ],
    )(xrow, wg_all, bg_all, be_all, we_all,
      wf, bfv, wcd, bcd, wk2, bk2, wout, bout)


def _gate_body(xrow_ref, wg_ref, bg_ref, idx_ref):
    logits = _dot_t(xrow_ref[...], wg_ref[0]) + bg_ref[0]
    iota = lax.broadcasted_iota(jnp.int32, logits.shape, 1)
    m1 = jnp.max(logits)
    i1 = jnp.min(jnp.where(logits >= m1, iota, 8))
    masked = jnp.where(iota == i1, -jnp.inf, logits)
    m2 = jnp.max(masked)
    i2 = jnp.min(jnp.where(masked >= m2, iota, 8))
    idx_ref[0] = i1.astype(jnp.int32)
    idx_ref[1] = i2.astype(jnp.int32)


def _gate(xrow, wg_all, bg_all, layer):
    return pl.pallas_call(
        _gate_body,
        grid=(1,),
        in_specs=[
            pl.BlockSpec((1, H), lambda t: (0, 0)),
            pl.BlockSpec((1, 8, H), lambda t: (layer, 0, 0)),
            pl.BlockSpec((1, 1, 8), lambda t: (layer, 0, 0)),
        ],
        out_shape=jax.ShapeDtypeStruct((2,), jnp.int32),
        out_specs=pl.BlockSpec(memory_space=pltpu.SMEM),
    )(xrow, wg_all, bg_all)


def _avg_body(idx_ref, we_ref, be_ref, xrow_ref, wavg_ref, bavg_ref,
              xnext_ref):
    k = pl.program_id(0)

    @pl.when(k == 0)
    def _():
        wavg_ref[...] = jnp.zeros_like(wavg_ref)
        bavg_ref[...] = jnp.zeros_like(bavg_ref)
        xnext_ref[...] = jnp.zeros_like(xnext_ref)

    w = we_ref[0, 0]
    b = be_ref[0, 0]
    wavg_ref[...] += 0.5 * w
    bavg_ref[...] += 0.5 * b
    xnext_ref[...] += 0.5 * (_dot_t(xrow_ref[...], w) + b)


def _avg(idx, we_all, be_all, xrow, layer):
    grid_spec = pltpu.PrefetchScalarGridSpec(
        num_scalar_prefetch=1,
        grid=(2,),
        in_specs=[
            pl.BlockSpec((1, 1, H, H),
                         lambda k, idx_ref: (layer, idx_ref[k], 0, 0)),
            pl.BlockSpec((1, 1, 1, H),
                         lambda k, idx_ref: (layer, idx_ref[k], 0, 0)),
            pl.BlockSpec((1, H), lambda k, idx_ref: (0, 0)),
        ],
        out_specs=[
            pl.BlockSpec((H, H), lambda k, idx_ref: (0, 0)),
            pl.BlockSpec((1, H), lambda k, idx_ref: (0, 0)),
            pl.BlockSpec((1, H), lambda k, idx_ref: (0, 0)),
        ],
    )
    return pl.pallas_call(
        _avg_body,
        grid_spec=grid_spec,
        out_shape=[
            jax.ShapeDtypeStruct((H, H), jnp.float32),
            jax.ShapeDtypeStruct((1, H), jnp.float32),
            jax.ShapeDtypeStruct((1, H), jnp.float32),
        ],
    )(idx, we_all, be_all, xrow)


def _compose_body(w0_ref, b0_ref, w1_ref, b1_ref, w2_ref, b2_ref,
                  wc_ref, bc_ref):
    w21 = _dot(w2_ref[...], w1_ref[...])
    wc_ref[...] = _dot(w21, w0_ref[...])
    b01 = _dot_t(b0_ref[...], w1_ref[...]) + b1_ref[...]
    bc_ref[...] = _dot_t(b01, w2_ref[...]) + b2_ref[...]


def _compose(w0, b0, w1, b1, w2, b2):
    return pl.pallas_call(
        _compose_body,
        out_shape=[
            jax.ShapeDtypeStruct((H, H), jnp.float32),
            jax.ShapeDtypeStruct((1, H), jnp.float32),
        ],
    )(w0, b0, w1, b1, w2, b2)


def _moe_qkv_body(x_ref, wc_ref, bc_ref, wqkv_ref, bqkv_ref,
                  xmoe_ref, qkv_ref):
    y = _dot_t(x_ref[...], wc_ref[...]) + bc_ref[...]
    xmoe_ref[...] = y
    z = _dot_t(y, wqkv_ref[...]) + bqkv_ref[...]
    zq = z[:, :H] * _QSCALE
    qkv_ref[:, :H] = zq.astype(jnp.bfloat16)
    qkv_ref[:, H:] = z[:, H:].astype(jnp.bfloat16)


def _moe_qkv(x, wc, bc, wqkv, bqkv):
    S = x.shape[0]
    full = lambda t: (0, 0)
    return pl.pallas_call(
        _moe_qkv_body,
        grid=(S // BLK,),
        in_specs=[
            pl.BlockSpec((BLK, H), lambda t: (t, 0)),
            pl.BlockSpec((H, H), full), pl.BlockSpec((1, H), full),
            pl.BlockSpec((3 * H, H), full), pl.BlockSpec((1, 3 * H), full),
        ],
        out_specs=[
            pl.BlockSpec((BLK, H), lambda t: (t, 0)),
            pl.BlockSpec((BLK, 3 * H), lambda t: (t, 0)),
        ],
        out_shape=[
            jax.ShapeDtypeStruct((S, H), jnp.float32),
            jax.ShapeDtypeStruct((S, 3 * H), jnp.bfloat16),
        ],
        compiler_params=pltpu.CompilerParams(
            dimension_semantics=("parallel",)),
    )(x, wc, bc, wqkv, bqkv)


BQ = 2048


def _attn_body(q_ref, k_ref, v_ref, o_ref):
    q = q_ref[...]
    k = k_ref[...]
    s = lax.dot_general(q, k, (((1,), (1,)), ((), ())),
                        preferred_element_type=jnp.float32)
    e = jnp.exp2(s)
    r = 1.0 / jnp.sum(e, axis=-1, keepdims=True)
    o_ref[...] = jnp.dot(e.astype(jnp.bfloat16), v_ref[...],
                         preferred_element_type=jnp.float32) * r


def _attention(qkv, S):
    return pl.pallas_call(
        _attn_body,
        grid=(NHEAD, S // BQ),
        in_specs=[
            pl.BlockSpec((BQ, DH), lambda h, t: (t, h)),
            pl.BlockSpec((S, DH), lambda h, t: (0, NHEAD + h)),
            pl.BlockSpec((S, DH), lambda h, t: (0, 2 * NHEAD + h)),
        ],
        out_specs=pl.BlockSpec((BQ, DH), lambda h, t: (t, h)),
        out_shape=jax.ShapeDtypeStruct((S, H), jnp.float32),
        compiler_params=pltpu.CompilerParams(
            dimension_semantics=("parallel", "parallel")),
    )(qkv, qkv, qkv)


def _ln(x, g, b):
    m = jnp.mean(x, axis=-1, keepdims=True)
    c = x - m
    v = jnp.mean(c * c, axis=-1, keepdims=True)
    return c * lax.rsqrt(v + 1e-5) * g + b


def _post_body(xmoe_ref, ao_ref, wo_ref, bo_ref, g1_ref, be1_ref,
               w1_ref, b1_ref, w2_ref, b2_ref, g2_ref, be2_ref,
               wa_ref, ba_ref, wk1_ref, bk1_ref, wb_ref, bb_ref, o_ref):
    t = _dot_t(ao_ref[...], wo_ref[...]) + bo_ref[...]
    x = _ln(xmoe_ref[...] + t, g1_ref[...], be1_ref[...])
    h = jnp.maximum(_dot_t(x, w1_ref[...]) + b1_ref[...], 0.0)
    f = _dot_t(h, w2_ref[...]) + b2_ref[...]
    x = _ln(x + f, g2_ref[...], be2_ref[...])
    x = _dot_t(x, wa_ref[...]) + ba_ref[...]
    h = jnp.maximum(_dot_t(x, wk1_ref[...]) + bk1_ref[...], 0.0)
    o_ref[...] = _dot_t(h, wb_ref[...]) + bb_ref[...]


def _post_tail(xmoe, ao, wo, bo, g1, be1, w1, b1, w2, b2, g2, be2,
               wa, ba, wk1, bk1, wb, bb):
    S = xmoe.shape[0]
    full = lambda t: (0, 0)
    wspec = pl.BlockSpec((H, H), full)
    bspec = pl.BlockSpec((1, H), full)
    blk = pl.BlockSpec((BLK, H), lambda t: (t, 0))
    return pl.pallas_call(
        _post_body,
        grid=(S // BLK,),
        in_specs=[
            blk, blk,
            wspec, bspec, bspec, bspec,
            pl.BlockSpec((2 * H, H), full), pl.BlockSpec((1, 2 * H), full),
            pl.BlockSpec((H, 2 * H), full), bspec,
            bspec, bspec,
            wspec, bspec, wspec, bspec, wspec, bspec,
        ],
        out_specs=blk,
        out_shape=jax.ShapeDtypeStruct((S, H), jnp.float32),
        compiler_params=pltpu.CompilerParams(
            dimension_semantics=("parallel",)),
    )(xmoe, ao, wo, bo, g1, be1, w1, b1, w2, b2, g2, be2,
      wa, ba, wk1, bk1, wb, bb)


def kernel(x, moe_We, moe_be, moe_Wg, moe_bg, Wqkv, bqkv, Wo, bo, ln1_g,
           ln1_b, W1, b1, W2, b2, ln2_g, ln2_b, Wf, bf, Wc, bc, Wk1, bk1,
           Wk2, bk2, Wout, bout):
    S = x.shape[0]
    row = lambda v: v.reshape(1, -1)

    wc_moe, bc_moe, wa, ba, wb, bb = _route(
        x[0:1], moe_Wg, moe_bg.reshape(3, 1, 8),
        moe_be.reshape(3, 8, 1, H), moe_We,
        Wf, row(bf), Wc, row(bc), Wk2, row(bk2), Wout, row(bout))

    xmoe, qkv = _moe_qkv(x, wc_moe, bc_moe, Wqkv, row(bqkv))
    ao = _attention(qkv, S)
    return _post_tail(xmoe, ao, Wo, row(bo), row(ln1_g), row(ln1_b),
                      W1, row(b1), W2, row(b2), row(ln2_g), row(ln2_b),
                      wa, ba, Wk1, row(bk1), wb, bb)


# speedup vs baseline: 1.0130x; 1.0009x over previous
import jax
import jax.numpy as jnp
from jax import lax
from jax.experimental import pallas as pl
from jax.experimental.pallas import tpu as pltpu

NHEAD = 4
H = 1024
DH = H // NHEAD
BLK = 512
_QSCALE = 1.4426950408889634 / (DH ** 0.5)


def _dot_t(a, w):
    return lax.dot_general(a, w, (((1,), (1,)), ((), ())),
                           preferred_element_type=jnp.float32)


def _dot(a, w):
    return jnp.dot(a, w, preferred_element_type=jnp.float32)


def _route_body(xrow_ref, wg_ref, bg_ref, be_ref, we_hbm,
                wf_ref, bfv_ref, wcd_ref, bcd_ref,
                wk2_ref, bk2_ref, wout_ref, bout_ref,
                wc_ref, bc_ref, wa_ref, ba_ref, wb_ref, bb_ref,
                w_a, w_b, sem_a, sem_b, xrow_scr):
    i = pl.program_id(0)

    @pl.when(i == 0)
    def _():
        xrow_scr[...] = xrow_ref[...]
        wa_ref[...] = _dot(wcd_ref[...], wf_ref[...])
        ba_ref[...] = _dot_t(bfv_ref[...], wcd_ref[...]) + bcd_ref[...]
        wb_ref[...] = _dot(wout_ref[...], wk2_ref[...])
        bb_ref[...] = _dot_t(bk2_ref[...], wout_ref[...]) + bout_ref[...]

    xr = xrow_scr[...]
    logits = _dot_t(xr, wg_ref[i]) + bg_ref[i]
    iota = lax.broadcasted_iota(jnp.int32, logits.shape, 1)
    m1 = jnp.max(logits)
    i1 = jnp.min(jnp.where(logits >= m1, iota, 8)).astype(jnp.int32)
    masked = jnp.where(iota == i1, -jnp.inf, logits)
    m2 = jnp.max(masked)
    i2 = jnp.min(jnp.where(masked >= m2, iota, 8)).astype(jnp.int32)

    cp_a = pltpu.make_async_copy(we_hbm.at[i, i1], w_a, sem_a)
    cp_b = pltpu.make_async_copy(we_hbm.at[i, i2], w_b, sem_b)
    cp_a.start()
    cp_b.start()
    cp_a.wait()
    cp_b.wait()

    wavg = 0.5 * (w_a[...] + w_b[...])
    bavg = 0.5 * (be_ref[i, i1] + be_ref[i, i2])

    @pl.when(i == 0)
    def _():
        wc_ref[...] = wavg
        bc_ref[...] = bavg

    @pl.when(i > 0)
    def _():
        wc_ref[...] = _dot(wavg, wc_ref[...])
        bc_ref[...] = _dot_t(bc_ref[...], wavg) + bavg

    xrow_scr[...] = _dot_t(xr, wavg) + bavg


def _route(xrow, wg_all, bg_all, be_all, we_all,
           wf, bfv, wcd, bcd, wk2, bk2, wout, bout):
    wspec = pl.BlockSpec((H, H), lambda i: (0, 0))
    bspec = pl.BlockSpec((1, H), lambda i: (0, 0))
    return pl.pallas_call(
        _route_body,
        grid=(3,),
        in_specs=[
            pl.BlockSpec((1, H), lambda i: (0, 0)),
            pl.BlockSpec((3, 8, H), lambda i: (0, 0, 0)),
            pl.BlockSpec((3, 1, 8), lambda i: (0, 0, 0)),
            pl.BlockSpec((3, 8, 1, H), lambda i: (0, 0, 0, 0)),
            pl.BlockSpec(memory_space=pl.ANY),
            wspec, bspec, wspec, bspec,
            wspec, bspec, wspec, bspec,
        ],
        out_specs=[
            wspec, bspec, wspec, bspec, wspec, bspec,
        ],
        out_shape=[
            jax.ShapeDtypeStruct((H, H), jnp.float32),
            jax.ShapeDtypeStruct((1, H), jnp.float32),
            jax.ShapeDtypeStruct((H, H), jnp.float32),
            jax.ShapeDtypeStruct((1, H), jnp.float32),
            jax.ShapeDtypeStruct((H, H), jnp.float32),
            jax.ShapeDtypeStruct((1, H), jnp.float32),
        ],
        scratch_shapes=[
            pltpu.VMEM((H, H), jnp.float32),
            pltpu.VMEM((H, H), jnp.float32),
            pltpu.SemaphoreType.DMA,
            pltpu.SemaphoreType.DMA,
            pltpu.VMEM((1, H), jnp.float32),
        ],
    )(xrow, wg_all, bg_all, be_all, we_all,
      wf, bfv, wcd, bcd, wk2, bk2, wout, bout)


def _moe_qkv_body(x_ref, wc_ref, bc_ref, wqkv_ref, bqkv_ref,
                  xmoe_ref, qkv_ref):
    y = _dot_t(x_ref[...], wc_ref[...]) + bc_ref[...]
    xmoe_ref[...] = y
    z = _dot_t(y, wqkv_ref[...]) + bqkv_ref[...]
    zq = z[:, :H] * _QSCALE
    qkv_ref[:, :H] = zq.astype(jnp.bfloat16)
    qkv_ref[:, H:] = z[:, H:].astype(jnp.bfloat16)


def _moe_qkv(x, wc, bc, wqkv, bqkv):
    S = x.shape[0]
    full = lambda t: (0, 0)
    return pl.pallas_call(
        _moe_qkv_body,
        grid=(S // BLK,),
        in_specs=[
            pl.BlockSpec((BLK, H), lambda t: (t, 0)),
            pl.BlockSpec((H, H), full), pl.BlockSpec((1, H), full),
            pl.BlockSpec((3 * H, H), full), pl.BlockSpec((1, 3 * H), full),
        ],
        out_specs=[
            pl.BlockSpec((BLK, H), lambda t: (t, 0)),
            pl.BlockSpec((BLK, 3 * H), lambda t: (t, 0)),
        ],
        out_shape=[
            jax.ShapeDtypeStruct((S, H), jnp.float32),
            jax.ShapeDtypeStruct((S, 3 * H), jnp.bfloat16),
        ],
        compiler_params=pltpu.CompilerParams(
            dimension_semantics=("parallel",)),
    )(x, wc, bc, wqkv, bqkv)


BQ = 2048


def _attn_body(q_ref, k_ref, v_ref, o_ref):
    q = q_ref[...]
    k = k_ref[...]
    s = lax.dot_general(q, k, (((1,), (1,)), ((), ())),
                        preferred_element_type=jnp.float32)
    e = jnp.exp2(s)
    r = 1.0 / jnp.sum(e, axis=-1, keepdims=True)
    o_ref[...] = jnp.dot(e.astype(jnp.bfloat16), v_ref[...],
                         preferred_element_type=jnp.float32) * r


def _attention(qkv, S):
    return pl.pallas_call(
        _attn_body,
        grid=(NHEAD, S // BQ),
        in_specs=[
            pl.BlockSpec((BQ, DH), lambda h, t: (t, h)),
            pl.BlockSpec((S, DH), lambda h, t: (0, NHEAD + h)),
            pl.BlockSpec((S, DH), lambda h, t: (0, 2 * NHEAD + h)),
        ],
        out_specs=pl.BlockSpec((BQ, DH), lambda h, t: (t, h)),
        out_shape=jax.ShapeDtypeStruct((S, H), jnp.float32),
        compiler_params=pltpu.CompilerParams(
            dimension_semantics=("parallel", "parallel")),
    )(qkv, qkv, qkv)


def _ln(x, g, b):
    m = jnp.mean(x, axis=-1, keepdims=True)
    c = x - m
    v = jnp.mean(c * c, axis=-1, keepdims=True)
    return c * lax.rsqrt(v + 1e-5) * g + b


def _post_body(xmoe_ref, ao_ref, wo_ref, bo_ref, g1_ref, be1_ref,
               w1_ref, b1_ref, w2_ref, b2_ref, g2_ref, be2_ref,
               wa_ref, ba_ref, wk1_ref, bk1_ref, wb_ref, bb_ref, o_ref):
    t = _dot_t(ao_ref[...], wo_ref[...]) + bo_ref[...]
    x = _ln(xmoe_ref[...] + t, g1_ref[...], be1_ref[...])
    h = jnp.maximum(_dot_t(x, w1_ref[...]) + b1_ref[...], 0.0)
    f = _dot_t(h, w2_ref[...]) + b2_ref[...]
    x = _ln(x + f, g2_ref[...], be2_ref[...])
    x = _dot_t(x, wa_ref[...]) + ba_ref[...]
    h = jnp.maximum(_dot_t(x, wk1_ref[...]) + bk1_ref[...], 0.0)
    o_ref[...] = _dot_t(h, wb_ref[...]) + bb_ref[...]


def _post_tail(xmoe, ao, wo, bo, g1, be1, w1, b1, w2, b2, g2, be2,
               wa, ba, wk1, bk1, wb, bb):
    S = xmoe.shape[0]
    full = lambda t: (0, 0)
    wspec = pl.BlockSpec((H, H), full)
    bspec = pl.BlockSpec((1, H), full)
    blk = pl.BlockSpec((BLK, H), lambda t: (t, 0))
    return pl.pallas_call(
        _post_body,
        grid=(S // BLK,),
        in_specs=[
            blk, blk,
            wspec, bspec, bspec, bspec,
            pl.BlockSpec((2 * H, H), full), pl.BlockSpec((1, 2 * H), full),
            pl.BlockSpec((H, 2 * H), full), bspec,
            bspec, bspec,
            wspec, bspec, wspec, bspec, wspec, bspec,
        ],
        out_specs=blk,
        out_shape=jax.ShapeDtypeStruct((S, H), jnp.float32),
        compiler_params=pltpu.CompilerParams(
            dimension_semantics=("parallel",)),
    )(xmoe, ao, wo, bo, g1, be1, w1, b1, w2, b2, g2, be2,
      wa, ba, wk1, bk1, wb, bb)


def kernel(x, moe_We, moe_be, moe_Wg, moe_bg, Wqkv, bqkv, Wo, bo, ln1_g,
           ln1_b, W1, b1, W2, b2, ln2_g, ln2_b, Wf, bf, Wc, bc, Wk1, bk1,
           Wk2, bk2, Wout, bout):
    S = x.shape[0]
    row = lambda v: v.reshape(1, -1)

    wc_moe, bc_moe, wa, ba, wb, bb = _route(
        x[0:1], moe_Wg, moe_bg.reshape(3, 1, 8),
        moe_be.reshape(3, 8, 1, H), moe_We,
        Wf, row(bf), Wc, row(bc), Wk2, row(bk2), Wout, row(bout))

    xmoe, qkv = _moe_qkv(x, wc_moe, bc_moe, Wqkv, row(bqkv))
    ao = _attention(qkv, S)
    return _post_tail(xmoe, ao, Wo, row(bo), row(ln1_g), row(ln1_b),
                      W1, row(b1), W2, row(b2), row(ln2_g), row(ln2_b),
                      wa, ba, Wk1, row(bk1), wb, bb)
